# baseline (speedup 1.0000x reference)
import functools

import jax
import jax.numpy as jnp
from jax import lax
from jax.experimental import pallas as pl
from jax.experimental.pallas import tpu as pltpu
from jax.experimental.pallas import tpu_sc as plsc


_K = 16
_NBLK = 256
_ROWS = 512
_G = 1


def _producer_body(xb_ref, xt_ref, w_ref, bias_ref,
                   distt_ref, m_ref, u_ref, v_ref, *, n_rows, n_ch):
    xb = xb_ref[0]
    xt = xt_ref[0]
    w = w_ref[...]
    wc = w[:, :n_ch]
    wd = w[:, n_ch:]
    n_nodes = xb.shape[1]
    n_sub = n_nodes // _NBLK

    ss_col = jnp.sum(xb * xb, axis=0, keepdims=True)
    g_mat = lax.dot_general(
        xt, xb, (((0,), (0,)), ((), ())),
        preferred_element_type=jnp.float32)
    dist = ss_col - 2.0 * g_mat
    distt_ref[0] = dist

    gt_mat = lax.dot_general(
        xb, xt, (((0,), (0,)), ((), ())),
        preferred_element_type=jnp.float32)
    ss_t = jnp.swapaxes(ss_col, 0, 1)
    dist_t = ss_t - 2.0 * gt_mat
    m_t = jnp.min(dist_t.reshape(_NBLK, n_sub, n_rows), axis=1)
    m_ref[0] = jnp.swapaxes(m_t, 0, 1)

    u_ref[0] = lax.dot_general(
        xt, wc - wd, (((0,), (1,)), ((), ())),
        preferred_element_type=jnp.float32) + bias_ref[...]
    v_ref[0] = lax.dot_general(
        xt, wd, (((0,), (1,)), ((), ())),
        preferred_element_type=jnp.float32)


def _tc_producer(x, W, b):
    batch, n_ch, n_nodes = x.shape
    n_out = W.shape[0]
    grid = (batch, n_nodes // _ROWS)
    return pl.pallas_call(
        functools.partial(_producer_body, n_rows=_ROWS, n_ch=n_ch),
        grid=grid,
        in_specs=[
            pl.BlockSpec((1, n_ch, n_nodes), lambda bi, ri: (bi, 0, 0)),
            pl.BlockSpec((1, n_ch, _ROWS), lambda bi, ri: (bi, 0, ri)),
            pl.BlockSpec((n_out, 2 * n_ch), lambda bi, ri: (0, 0)),
            pl.BlockSpec((1, n_out), lambda bi, ri: (0, 0)),
        ],
        out_specs=[
            pl.BlockSpec((1, _ROWS, n_nodes), lambda bi, ri: (bi, ri, 0)),
            pl.BlockSpec((1, _ROWS, _NBLK), lambda bi, ri: (bi, ri, 0)),
            pl.BlockSpec((1, _ROWS, n_out), lambda bi, ri: (bi, ri, 0)),
            pl.BlockSpec((1, _ROWS, n_out), lambda bi, ri: (bi, ri, 0)),
        ],
        out_shape=[
            jax.ShapeDtypeStruct((batch, n_nodes, n_nodes), jnp.float32),
            jax.ShapeDtypeStruct((batch, n_nodes, _NBLK), jnp.float32),
            jax.ShapeDtypeStruct((batch, n_nodes, n_out), jnp.float32),
            jax.ShapeDtypeStruct((batch, n_nodes, n_out), jnp.float32),
        ],
        compiler_params=pltpu.CompilerParams(
            dimension_semantics=("parallel", "arbitrary"),
        ),
    )(x, x, W, b.reshape(1, n_out))


def _min_merge(ka, va, kb, vb):
    rk = jnp.flip(kb)
    rv = jnp.flip(vb)
    take_a = ka <= rk
    mk = jnp.where(take_a, ka, rk)
    mv = jnp.where(take_a, va, rv)
    return plsc.sort_key_val(mk, mv)


def _tournament(keys, vals):
    pairs = [plsc.sort_key_val(k, v) for k, v in zip(keys, vals)]
    while len(pairs) > 1:
        nxt = []
        for i in range(0, len(pairs), 2):
            (ka, va), (kb, vb) = pairs[i], pairs[i + 1]
            nxt.append(_min_merge(ka, va, kb, vb))
        pairs = nxt
    return pairs[0]


def _sc_consumer(distt, mblk, u, v):
    batch, n_nodes, n_out = u.shape
    total = batch * n_nodes
    distt_flat = distt.reshape(total * _NBLK, _K)
    m_flat = mblk.reshape(total, _NBLK)
    u_flat = u.reshape(total, n_out)
    v_flat = v.reshape(total, n_out)

    info = plsc.get_sparse_core_info()
    n_workers = info.num_cores * info.num_subcores
    rows_per_w = total // n_workers
    n_groups = rows_per_w // _G
    n_iters = n_groups + 2
    mesh = plsc.VectorSubcoreMesh(core_axis_name="c", subcore_axis_name="s")

    @functools.partial(
        pl.kernel, mesh=mesh,
        out_type=jax.ShapeDtypeStruct((total, n_out), jnp.float32),
        compiler_params=pltpu.CompilerParams(
            needs_layout_passes=False, use_tc_tiling_on_sc=False),
        scratch_types=[
            pltpu.VMEM((2, _G, _NBLK), jnp.float32),
            pltpu.VMEM((2, _G, n_out), jnp.float32),
            pltpu.VMEM((2, _G, _K, _K), jnp.float32),
            pltpu.VMEM((2, _G, _K, n_out), jnp.float32),
            pltpu.VMEM((2, _G, n_out), jnp.float32),
            pltpu.VMEM((2, _G, _K), jnp.int32),
            pltpu.SemaphoreType.DMA,
            pltpu.SemaphoreType.DMA,
            pltpu.SemaphoreType.DMA,
            pltpu.SemaphoreType.DMA,
            pltpu.SemaphoreType.DMA,
            pltpu.SemaphoreType.DMA,
            pltpu.SemaphoreType.DMA,
            pltpu.SemaphoreType.DMA,
            pltpu.SemaphoreType.DMA,
            pltpu.SemaphoreType.DMA,
        ],
    )
    def consumer(distt_hbm, m_hbm, u_hbm, v_hbm, out_hbm,
                 m_buf, u_buf, chunk_buf, v_buf, out_buf, blk_store,
                 m_sem0, m_sem1, u_sem0, u_sem1, c_sem0, c_sem1,
                 v_sem0, v_sem1, o_sem0, o_sem1):
        m_sems = (m_sem0, m_sem1)
        u_sems = (u_sem0, u_sem1)
        c_sems = (c_sem0, c_sem1)
        v_sems = (v_sem0, v_sem1)
        o_sems = (o_sem0, o_sem1)

        wid = lax.axis_index("c") * info.num_subcores + lax.axis_index("s")
        base = wid * rows_per_w
        batch_base = (base // n_nodes) * n_nodes

        lane = lax.iota(jnp.int32, _K)
        leaf_ids = [lane + _K * c for c in range(_K)]
        pos_ids = [lane + _K * c for c in range(_K)]

        def fire_m(g, par):
            pltpu.make_async_copy(
                m_hbm.at[pl.ds(base + g * _G, _G)],
                m_buf.at[par], m_sems[par]).start()

        def fire_u(g, par):
            pltpu.make_async_copy(
                u_hbm.at[pl.ds(base + g * _G, _G)],
                u_buf.at[par], u_sems[par]).start()

        def stage_a(g, par):
            pltpu.make_async_copy(
                m_hbm.at[pl.ds(0, _G)], m_buf.at[par], m_sems[par]).wait()
            row0 = base + g * _G
            for r in range(_G):
                keys = [m_buf[par, r, pl.ds(c * _K, _K)] for c in range(_K)]
                _, blk = _tournament(keys, leaf_ids)
                blk_store[par, r, :] = blk
                chunk_ids = blk + (row0 + r) * _NBLK
                pltpu.make_async_copy(
                    distt_hbm.at[chunk_ids],
                    chunk_buf.at[par, r], c_sems[par]).start()

        def stage_b(g, par):
            for r in range(_G):
                pltpu.make_async_copy(
                    distt_hbm.at[pl.ds(0, _K)],
                    chunk_buf.at[par, r], c_sems[par]).wait()
            for r in range(_G):
                keys = [chunk_buf[par, r, c, :] for c in range(_K)]
                _, pos = _tournament(keys, pos_ids)
                src_blk = plsc.load_gather(
                    blk_store.at[par, r], [lax.shift_right_logical(pos, 4)])
                cols = _K * src_blk + (pos & 15)
                vids = cols + batch_base
                pltpu.make_async_copy(
                    v_hbm.at[vids], v_buf.at[par, r], v_sems[par]).start()

        def stage_c(g, par):
            pltpu.make_async_copy(
                u_hbm.at[pl.ds(0, _G)], u_buf.at[par], u_sems[par]).wait()

            @pl.when(g >= 2)
            def _():
                pltpu.make_async_copy(
                    out_buf.at[par], out_hbm.at[pl.ds(0, _G)],
                    o_sems[par]).wait()

            for r in range(_G):
                pltpu.make_async_copy(
                    v_hbm.at[pl.ds(0, _K)], v_buf.at[par, r],
                    v_sems[par]).wait()
            for r in range(_G):
                for l in range(n_out // _K):
                    acc = v_buf[par, r, 0, pl.ds(l * _K, _K)]
                    for q in range(1, _K):
                        acc = jnp.maximum(
                            acc, v_buf[par, r, q, pl.ds(l * _K, _K)])
                    acc = acc + u_buf[par, r, pl.ds(l * _K, _K)]
                    out_buf[par, r, pl.ds(l * _K, _K)] = jnp.maximum(acc, 0.0)
            pltpu.make_async_copy(
                out_buf.at[par], out_hbm.at[pl.ds(base + g * _G, _G)],
                o_sems[par]).start()

        fire_m(0, 0)

        def half_iter(it, k):
            @pl.when(it + 1 < n_groups)
            def _():
                fire_m(it + 1, 1 - k)

            @pl.when((it >= 1) & (it - 1 < n_groups))
            def _():
                fire_u(it - 1, 1 - k)

            @pl.when(it < n_groups)
            def _():
                stage_a(it, k)

            @pl.when((it >= 1) & (it - 1 < n_groups))
            def _():
                stage_b(it - 1, 1 - k)

            @pl.when(it >= 2)
            def _():
                stage_c(it - 2, k)

        def body(i, carry):
            half_iter(2 * i, 0)
            half_iter(2 * i + 1, 1)
            return carry

        lax.fori_loop(0, (n_iters + 1) // 2, body, 0)

        pltpu.make_async_copy(
            out_buf.at[0], out_hbm.at[pl.ds(0, _G)], o_sems[0]).wait()
        pltpu.make_async_copy(
            out_buf.at[1], out_hbm.at[pl.ds(0, _G)], o_sems[1]).wait()

    return consumer(distt_flat, m_flat, u_flat, v_flat)


@jax.jit
def kernel(x, W, b):
    batch, n_ch, n_nodes = x.shape
    distt, mblk, u, v = _tc_producer(x, W, b)
    out_flat = _sc_consumer(distt, mblk, u, v)
    return jnp.transpose(out_flat.reshape(batch, n_nodes, -1), (0, 2, 1))


# speedup vs baseline: 1.1867x; 1.1867x over previous
import functools

import jax
import jax.numpy as jnp
from jax import lax
from jax.experimental import pallas as pl
from jax.experimental.pallas import tpu as pltpu
from jax.experimental.pallas import tpu_sc as plsc


_K = 16
_NBLK = 256
_ROWS = 512
_G = 2


def _producer_body(xb_ref, xt_ref, w_ref, bias_ref,
                   distt_ref, m_ref, u_ref, v_ref, *, n_rows, n_ch):
    xb = xb_ref[0]
    xt = xt_ref[0]
    w = w_ref[...]
    wc = w[:, :n_ch]
    wd = w[:, n_ch:]
    n_nodes = xb.shape[1]
    n_sub = n_nodes // _NBLK

    ss_col = jnp.sum(xb * xb, axis=0, keepdims=True)
    g_mat = lax.dot_general(
        xt, xb, (((0,), (0,)), ((), ())),
        preferred_element_type=jnp.float32)
    dist = ss_col - 2.0 * g_mat
    distt_ref[0] = dist

    gt_mat = lax.dot_general(
        xb, xt, (((0,), (0,)), ((), ())),
        preferred_element_type=jnp.float32)
    ss_t = jnp.swapaxes(ss_col, 0, 1)
    dist_t = ss_t - 2.0 * gt_mat
    m_t = jnp.min(dist_t.reshape(_NBLK, n_sub, n_rows), axis=1)
    m_ref[0] = jnp.swapaxes(m_t, 0, 1)

    u_ref[0] = lax.dot_general(
        xt, wc - wd, (((0,), (1,)), ((), ())),
        preferred_element_type=jnp.float32) + bias_ref[...]
    v_ref[0] = lax.dot_general(
        xt, wd, (((0,), (1,)), ((), ())),
        preferred_element_type=jnp.float32)


def _tc_producer(x, W, b):
    batch, n_ch, n_nodes = x.shape
    n_out = W.shape[0]
    grid = (batch, n_nodes // _ROWS)
    return pl.pallas_call(
        functools.partial(_producer_body, n_rows=_ROWS, n_ch=n_ch),
        grid=grid,
        in_specs=[
            pl.BlockSpec((1, n_ch, n_nodes), lambda bi, ri: (bi, 0, 0)),
            pl.BlockSpec((1, n_ch, _ROWS), lambda bi, ri: (bi, 0, ri)),
            pl.BlockSpec((n_out, 2 * n_ch), lambda bi, ri: (0, 0)),
            pl.BlockSpec((1, n_out), lambda bi, ri: (0, 0)),
        ],
        out_specs=[
            pl.BlockSpec((1, _ROWS, n_nodes), lambda bi, ri: (bi, ri, 0)),
            pl.BlockSpec((1, _ROWS, _NBLK), lambda bi, ri: (bi, ri, 0)),
            pl.BlockSpec((1, _ROWS, n_out), lambda bi, ri: (bi, ri, 0)),
            pl.BlockSpec((1, _ROWS, n_out), lambda bi, ri: (bi, ri, 0)),
        ],
        out_shape=[
            jax.ShapeDtypeStruct((batch, n_nodes, n_nodes), jnp.float32),
            jax.ShapeDtypeStruct((batch, n_nodes, _NBLK), jnp.float32),
            jax.ShapeDtypeStruct((batch, n_nodes, n_out), jnp.float32),
            jax.ShapeDtypeStruct((batch, n_nodes, n_out), jnp.float32),
        ],
        compiler_params=pltpu.CompilerParams(
            dimension_semantics=("parallel", "arbitrary"),
        ),
    )(x, x, W, b.reshape(1, n_out))


def _min_merge(ka, va, kb, vb):
    rk = jnp.flip(kb)
    rv = jnp.flip(vb)
    take_a = ka <= rk
    mk = jnp.where(take_a, ka, rk)
    mv = jnp.where(take_a, va, rv)
    return plsc.sort_key_val(mk, mv)


def _tournament(keys, vals):
    pairs = [plsc.sort_key_val(k, v) for k, v in zip(keys, vals)]
    while len(pairs) > 1:
        nxt = []
        for i in range(0, len(pairs), 2):
            (ka, va), (kb, vb) = pairs[i], pairs[i + 1]
            nxt.append(_min_merge(ka, va, kb, vb))
        pairs = nxt
    return pairs[0]


def _sc_consumer(distt, mblk, u, v):
    batch, n_nodes, n_out = u.shape
    total = batch * n_nodes
    distt_flat = distt.reshape(total * _NBLK, _K)
    m_flat = mblk.reshape(total, _NBLK)
    u_flat = u.reshape(total, n_out)
    v_flat = v.reshape(total, n_out)

    info = plsc.get_sparse_core_info()
    n_workers = info.num_cores * info.num_subcores
    rows_per_w = total // n_workers
    n_groups = rows_per_w // _G
    n_iters = n_groups + 2
    mesh = plsc.VectorSubcoreMesh(core_axis_name="c", subcore_axis_name="s")

    @functools.partial(
        pl.kernel, mesh=mesh,
        out_type=jax.ShapeDtypeStruct((total, n_out), jnp.float32),
        compiler_params=pltpu.CompilerParams(
            needs_layout_passes=False, use_tc_tiling_on_sc=False),
        scratch_types=[
            pltpu.VMEM((2, _G, _NBLK), jnp.float32),
            pltpu.VMEM((2, _G, n_out), jnp.float32),
            pltpu.VMEM((2, _G, _K, _K), jnp.float32),
            pltpu.VMEM((2, _G, _K, n_out), jnp.float32),
            pltpu.VMEM((2, _G, n_out), jnp.float32),
            pltpu.VMEM((2, _G, _K), jnp.int32),
            pltpu.SemaphoreType.DMA,
            pltpu.SemaphoreType.DMA,
            pltpu.SemaphoreType.DMA,
            pltpu.SemaphoreType.DMA,
            pltpu.SemaphoreType.DMA,
            pltpu.SemaphoreType.DMA,
            pltpu.SemaphoreType.DMA,
            pltpu.SemaphoreType.DMA,
            pltpu.SemaphoreType.DMA,
            pltpu.SemaphoreType.DMA,
        ],
    )
    def consumer(distt_hbm, m_hbm, u_hbm, v_hbm, out_hbm,
                 m_buf, u_buf, chunk_buf, v_buf, out_buf, blk_store,
                 m_sem0, m_sem1, u_sem0, u_sem1, c_sem0, c_sem1,
                 v_sem0, v_sem1, o_sem0, o_sem1):
        m_sems = (m_sem0, m_sem1)
        u_sems = (u_sem0, u_sem1)
        c_sems = (c_sem0, c_sem1)
        v_sems = (v_sem0, v_sem1)
        o_sems = (o_sem0, o_sem1)

        wid = lax.axis_index("c") * info.num_subcores + lax.axis_index("s")
        base = wid * rows_per_w
        batch_base = (base // n_nodes) * n_nodes

        lane = lax.iota(jnp.int32, _K)
        leaf_ids = [lane + _K * c for c in range(_K)]
        pos_ids = [lane + _K * c for c in range(_K)]

        def fire_m(g, par):
            pltpu.make_async_copy(
                m_hbm.at[pl.ds(base + g * _G, _G)],
                m_buf.at[par], m_sems[par]).start()

        def fire_u(g, par):
            pltpu.make_async_copy(
                u_hbm.at[pl.ds(base + g * _G, _G)],
                u_buf.at[par], u_sems[par]).start()

        def stage_a(g, par):
            pltpu.make_async_copy(
                m_hbm.at[pl.ds(0, _G)], m_buf.at[par], m_sems[par]).wait()
            row0 = base + g * _G
            for r in range(_G):
                keys = [m_buf[par, r, pl.ds(c * _K, _K)] for c in range(_K)]
                _, blk = _tournament(keys, leaf_ids)
                blk_store[par, r, :] = blk
                chunk_ids = blk + (row0 + r) * _NBLK
                pltpu.make_async_copy(
                    distt_hbm.at[chunk_ids],
                    chunk_buf.at[par, r], c_sems[par]).start()

        def stage_b(g, par):
            for r in range(_G):
                pltpu.make_async_copy(
                    distt_hbm.at[pl.ds(0, _K)],
                    chunk_buf.at[par, r], c_sems[par]).wait()
            for r in range(_G):
                keys = [chunk_buf[par, r, c, :] for c in range(_K)]
                _, pos = _tournament(keys, pos_ids)
                src_blk = plsc.load_gather(
                    blk_store.at[par, r], [lax.shift_right_logical(pos, 4)])
                cols = _K * src_blk + (pos & 15)
                vids = cols + batch_base
                pltpu.make_async_copy(
                    v_hbm.at[vids], v_buf.at[par, r], v_sems[par]).start()

        def stage_c(g, par):
            pltpu.make_async_copy(
                u_hbm.at[pl.ds(0, _G)], u_buf.at[par], u_sems[par]).wait()

            @pl.when(g >= 2)
            def _():
                pltpu.make_async_copy(
                    out_buf.at[par], out_hbm.at[pl.ds(0, _G)],
                    o_sems[par]).wait()

            for r in range(_G):
                pltpu.make_async_copy(
                    v_hbm.at[pl.ds(0, _K)], v_buf.at[par, r],
                    v_sems[par]).wait()
            for r in range(_G):
                for l in range(n_out // _K):
                    acc = v_buf[par, r, 0, pl.ds(l * _K, _K)]
                    for q in range(1, _K):
                        acc = jnp.maximum(
                            acc, v_buf[par, r, q, pl.ds(l * _K, _K)])
                    acc = acc + u_buf[par, r, pl.ds(l * _K, _K)]
                    out_buf[par, r, pl.ds(l * _K, _K)] = jnp.maximum(acc, 0.0)
            pltpu.make_async_copy(
                out_buf.at[par], out_hbm.at[pl.ds(base + g * _G, _G)],
                o_sems[par]).start()

        fire_m(0, 0)

        def half_iter(it, k):
            @pl.when(it + 1 < n_groups)
            def _():
                fire_m(it + 1, 1 - k)

            @pl.when((it >= 1) & (it - 1 < n_groups))
            def _():
                fire_u(it - 1, 1 - k)

            @pl.when(it < n_groups)
            def _():
                stage_a(it, k)

            @pl.when((it >= 1) & (it - 1 < n_groups))
            def _():
                stage_b(it - 1, 1 - k)

            @pl.when(it >= 2)
            def _():
                stage_c(it - 2, k)

        def body(i, carry):
            half_iter(2 * i, 0)
            half_iter(2 * i + 1, 1)
            return carry

        lax.fori_loop(0, (n_iters + 1) // 2, body, 0)

        pltpu.make_async_copy(
            out_buf.at[0], out_hbm.at[pl.ds(0, _G)], o_sems[0]).wait()
        pltpu.make_async_copy(
            out_buf.at[1], out_hbm.at[pl.ds(0, _G)], o_sems[1]).wait()

    return consumer(distt_flat, m_flat, u_flat, v_flat)


@jax.jit
def kernel(x, W, b):
    batch, n_ch, n_nodes = x.shape
    distt, mblk, u, v = _tc_producer(x, W, b)
    out_flat = _sc_consumer(distt, mblk, u, v)
    return jnp.transpose(out_flat.reshape(batch, n_nodes, -1), (0, 2, 1))


# speedup vs baseline: 1.6074x; 1.3544x over previous
import functools

import jax
import jax.numpy as jnp
from jax import lax
from jax.experimental import pallas as pl
from jax.experimental.pallas import tpu as pltpu
from jax.experimental.pallas import tpu_sc as plsc


_K = 16
_NBLK = 256
_ROWS = 512
_G = 2


def _producer_body(xb_ref, xt_ref, w_ref, bias_ref,
                   distt_ref, m_ref, u_ref, v_ref, *, n_rows, n_ch):
    xb = xb_ref[0]
    xt = xt_ref[0]
    w = w_ref[...]
    wc = w[:, :n_ch]
    wd = w[:, n_ch:]
    n_nodes = xb.shape[1]
    n_sub = n_nodes // _NBLK

    ss_col = jnp.sum(xb * xb, axis=0, keepdims=True)
    g_mat = lax.dot_general(
        xt, xb, (((0,), (0,)), ((), ())),
        preferred_element_type=jnp.float32)
    dist = ss_col - 2.0 * g_mat
    distt_ref[...] = jnp.swapaxes(
        dist.reshape(n_rows // 8, 8, n_nodes // 128, 128), 1, 2)

    gt_mat = lax.dot_general(
        xb, xt, (((0,), (0,)), ((), ())),
        preferred_element_type=jnp.float32)
    ss_t = jnp.swapaxes(ss_col, 0, 1)
    dist_t = ss_t - 2.0 * gt_mat
    m_t = jnp.min(dist_t.reshape(_NBLK, n_sub, n_rows), axis=1)
    m_ref[0] = jnp.swapaxes(m_t, 0, 1)

    u_ref[0] = lax.dot_general(
        xt, wc - wd, (((0,), (1,)), ((), ())),
        preferred_element_type=jnp.float32) + bias_ref[...]
    v_ref[0] = lax.dot_general(
        xt, wd, (((0,), (1,)), ((), ())),
        preferred_element_type=jnp.float32)


def _tc_producer(x, W, b):
    batch, n_ch, n_nodes = x.shape
    n_out = W.shape[0]
    grid = (batch, n_nodes // _ROWS)
    return pl.pallas_call(
        functools.partial(_producer_body, n_rows=_ROWS, n_ch=n_ch),
        grid=grid,
        in_specs=[
            pl.BlockSpec((1, n_ch, n_nodes), lambda bi, ri: (bi, 0, 0)),
            pl.BlockSpec((1, n_ch, _ROWS), lambda bi, ri: (bi, 0, ri)),
            pl.BlockSpec((n_out, 2 * n_ch), lambda bi, ri: (0, 0)),
            pl.BlockSpec((1, n_out), lambda bi, ri: (0, 0)),
        ],
        out_specs=[
            pl.BlockSpec((_ROWS // 8, n_nodes // 128, 8, 128),
                         lambda bi, ri: (bi * (n_nodes // _ROWS) + ri,
                                         0, 0, 0)),
            pl.BlockSpec((1, _ROWS, _NBLK), lambda bi, ri: (bi, ri, 0)),
            pl.BlockSpec((1, _ROWS, n_out), lambda bi, ri: (bi, ri, 0)),
            pl.BlockSpec((1, _ROWS, n_out), lambda bi, ri: (bi, ri, 0)),
        ],
        out_shape=[
            jax.ShapeDtypeStruct(
                (batch * n_nodes // 8, n_nodes // 128, 8, 128), jnp.float32),
            jax.ShapeDtypeStruct((batch, n_nodes, _NBLK), jnp.float32),
            jax.ShapeDtypeStruct((batch, n_nodes, n_out), jnp.float32),
            jax.ShapeDtypeStruct((batch, n_nodes, n_out), jnp.float32),
        ],
        compiler_params=pltpu.CompilerParams(
            dimension_semantics=("parallel", "arbitrary"),
        ),
    )(x, x, W, b.reshape(1, n_out))


def _min_merge(ka, va, kb, vb):
    rk = jnp.flip(kb)
    rv = jnp.flip(vb)
    take_a = ka <= rk
    mk = jnp.where(take_a, ka, rk)
    mv = jnp.where(take_a, va, rv)
    return plsc.sort_key_val(mk, mv)


def _tournament(keys, vals):
    pairs = [plsc.sort_key_val(k, v) for k, v in zip(keys, vals)]
    while len(pairs) > 1:
        nxt = []
        for i in range(0, len(pairs), 2):
            (ka, va), (kb, vb) = pairs[i], pairs[i + 1]
            nxt.append(_min_merge(ka, va, kb, vb))
        pairs = nxt
    return pairs[0]


def _sc_consumer(distt, mblk, u, v):
    batch, n_nodes, n_out = u.shape
    total = batch * n_nodes
    distt_flat = distt.reshape(total * _NBLK, _K)
    m_flat = mblk.reshape(total, _NBLK)
    u_flat = u.reshape(total, n_out)
    v_flat = v.reshape(total, n_out)

    info = plsc.get_sparse_core_info()
    n_workers = info.num_cores * info.num_subcores
    rows_per_w = total // n_workers
    n_groups = rows_per_w // _G
    n_iters = n_groups + 2
    mesh = plsc.VectorSubcoreMesh(core_axis_name="c", subcore_axis_name="s")

    @functools.partial(
        pl.kernel, mesh=mesh,
        out_type=jax.ShapeDtypeStruct((total, n_out), jnp.float32),
        compiler_params=pltpu.CompilerParams(
            needs_layout_passes=False, use_tc_tiling_on_sc=False),
        scratch_types=[
            pltpu.VMEM((2, _G, _NBLK), jnp.float32),
            pltpu.VMEM((2, _G, n_out), jnp.float32),
            pltpu.VMEM((2, _G, _K, _K), jnp.float32),
            pltpu.VMEM((2, _G, _K, n_out), jnp.float32),
            pltpu.VMEM((2, _G, n_out), jnp.float32),
            pltpu.VMEM((2, _G, _K), jnp.int32),
            pltpu.SemaphoreType.DMA,
            pltpu.SemaphoreType.DMA,
            pltpu.SemaphoreType.DMA,
            pltpu.SemaphoreType.DMA,
            pltpu.SemaphoreType.DMA,
            pltpu.SemaphoreType.DMA,
            pltpu.SemaphoreType.DMA,
            pltpu.SemaphoreType.DMA,
            pltpu.SemaphoreType.DMA,
            pltpu.SemaphoreType.DMA,
        ],
    )
    def consumer(distt_hbm, m_hbm, u_hbm, v_hbm, out_hbm,
                 m_buf, u_buf, chunk_buf, v_buf, out_buf, blk_store,
                 m_sem0, m_sem1, u_sem0, u_sem1, c_sem0, c_sem1,
                 v_sem0, v_sem1, o_sem0, o_sem1):
        m_sems = (m_sem0, m_sem1)
        u_sems = (u_sem0, u_sem1)
        c_sems = (c_sem0, c_sem1)
        v_sems = (v_sem0, v_sem1)
        o_sems = (o_sem0, o_sem1)

        wid = lax.axis_index("c") * info.num_subcores + lax.axis_index("s")
        base = wid * rows_per_w
        batch_base = (base // n_nodes) * n_nodes

        lane = lax.iota(jnp.int32, _K)
        leaf_ids = [lane + _K * c for c in range(_K)]
        pos_ids = [lane + _K * c for c in range(_K)]

        def fire_m(g, par):
            pltpu.make_async_copy(
                m_hbm.at[pl.ds(base + g * _G, _G)],
                m_buf.at[par], m_sems[par]).start()

        def fire_u(g, par):
            pltpu.make_async_copy(
                u_hbm.at[pl.ds(base + g * _G, _G)],
                u_buf.at[par], u_sems[par]).start()

        def stage_a(g, par):
            pltpu.make_async_copy(
                m_hbm.at[pl.ds(0, _G)], m_buf.at[par], m_sems[par]).wait()
            row0 = base + g * _G
            for r in range(_G):
                keys = [m_buf[par, r, pl.ds(c * _K, _K)] for c in range(_K)]
                _, blk = _tournament(keys, leaf_ids)
                blk_store[par, r, :] = blk
                row = row0 + r
                rp = (row // 8) * 2048 + (row % 8) * 8
                chunk_ids = rp + (blk // 8) * 64 + (blk % 8)
                pltpu.make_async_copy(
                    distt_hbm.at[chunk_ids],
                    chunk_buf.at[par, r], c_sems[par]).start()

        def stage_b(g, par):
            for r in range(_G):
                pltpu.make_async_copy(
                    distt_hbm.at[pl.ds(0, _K)],
                    chunk_buf.at[par, r], c_sems[par]).wait()
            for r in range(_G):
                keys = [chunk_buf[par, r, c, :] for c in range(_K)]
                _, pos = _tournament(keys, pos_ids)
                src_blk = plsc.load_gather(
                    blk_store.at[par, r], [lax.shift_right_logical(pos, 4)])
                cols = _K * src_blk + (pos & 15)
                vids = cols + batch_base
                pltpu.make_async_copy(
                    v_hbm.at[vids], v_buf.at[par, r], v_sems[par]).start()

        def stage_c(g, par):
            pltpu.make_async_copy(
                u_hbm.at[pl.ds(0, _G)], u_buf.at[par], u_sems[par]).wait()

            @pl.when(g >= 2)
            def _():
                pltpu.make_async_copy(
                    out_buf.at[par], out_hbm.at[pl.ds(0, _G)],
                    o_sems[par]).wait()

            for r in range(_G):
                pltpu.make_async_copy(
                    v_hbm.at[pl.ds(0, _K)], v_buf.at[par, r],
                    v_sems[par]).wait()
            for r in range(_G):
                for l in range(n_out // _K):
                    acc = v_buf[par, r, 0, pl.ds(l * _K, _K)]
                    for q in range(1, _K):
                        acc = jnp.maximum(
                            acc, v_buf[par, r, q, pl.ds(l * _K, _K)])
                    acc = acc + u_buf[par, r, pl.ds(l * _K, _K)]
                    out_buf[par, r, pl.ds(l * _K, _K)] = jnp.maximum(acc, 0.0)
            pltpu.make_async_copy(
                out_buf.at[par], out_hbm.at[pl.ds(base + g * _G, _G)],
                o_sems[par]).start()

        fire_m(0, 0)

        def half_iter(it, k):
            @pl.when(it + 1 < n_groups)
            def _():
                fire_m(it + 1, 1 - k)

            @pl.when((it >= 1) & (it - 1 < n_groups))
            def _():
                fire_u(it - 1, 1 - k)

            @pl.when(it < n_groups)
            def _():
                stage_a(it, k)

            @pl.when((it >= 1) & (it - 1 < n_groups))
            def _():
                stage_b(it - 1, 1 - k)

            @pl.when(it >= 2)
            def _():
                stage_c(it - 2, k)

        def body(i, carry):
            half_iter(2 * i, 0)
            half_iter(2 * i + 1, 1)
            return carry

        lax.fori_loop(0, (n_iters + 1) // 2, body, 0)

        pltpu.make_async_copy(
            out_buf.at[0], out_hbm.at[pl.ds(0, _G)], o_sems[0]).wait()
        pltpu.make_async_copy(
            out_buf.at[1], out_hbm.at[pl.ds(0, _G)], o_sems[1]).wait()

    return consumer(distt_flat, m_flat, u_flat, v_flat)


@jax.jit
def kernel(x, W, b):
    batch, n_ch, n_nodes = x.shape
    distt, mblk, u, v = _tc_producer(x, W, b)
    out_flat = _sc_consumer(distt, mblk, u, v)
    return jnp.transpose(out_flat.reshape(batch, n_nodes, -1), (0, 2, 1))
